# baseline (speedup 1.0000x reference)
import functools
import math

import jax
import jax.numpy as jnp
from jax import lax
from jax.experimental import pallas as pl
from jax.experimental.pallas import tpu as pltpu
from jax.experimental.pallas import tpu_sc as plsc

VOCAB = 100000
EMBED_DIM = 2048
BATCH = 4
SEQ_LEN = 4096

NC, NS, LANES = 2, 16, 16
NW = NC * NS
PW = SEQ_LEN // NW
CHUNK = 16
STEPS = (PW // CHUNK) * BATCH
VECS_PER_ROW = EMBED_DIM // LANES


def _sinusoidal_pe(seq_len: int, d: int) -> jnp.ndarray:
    pos = jnp.arange(seq_len, dtype=jnp.float32)[:, None]
    div = jnp.exp(jnp.arange(0, d, 2, dtype=jnp.float32) * (-math.log(10000.0) / d))
    pe = jnp.zeros((seq_len, d), dtype=jnp.float32)
    pe = pe.at[:, 0::2].set(jnp.sin(pos * div))
    pe = pe.at[:, 1::2].set(jnp.cos(pos * div))
    return pe


def _make_sc_kernel():
    mesh = plsc.VectorSubcoreMesh(
        core_axis_name="c", subcore_axis_name="s",
        num_cores=NC, num_subcores=NS,
    )

    @functools.partial(
        pl.kernel,
        out_type=jax.ShapeDtypeStruct((BATCH, SEQ_LEN, EMBED_DIM), jnp.float32),
        mesh=mesh,
        scratch_types=[
            pltpu.VMEM((BATCH, PW), jnp.int32),
            pltpu.VMEM((CHUNK, EMBED_DIM), jnp.float32),
            pltpu.VMEM((2, CHUNK, EMBED_DIM), jnp.float32),
            pltpu.SemaphoreType.DMA,
            pltpu.SemaphoreType.DMA,
            pltpu.SemaphoreType.DMA,
            pltpu.SemaphoreType.DMA,
        ],
    )
    def body(x_hbm, pe_hbm, table_hbm, out_hbm, idx_v, pe_v, row_vv,
             isem, gsem, ssem, psem):
        wid = lax.axis_index("s") * NC + lax.axis_index("c")
        pos0 = wid * PW

        for b in range(BATCH):
            pltpu.async_copy(x_hbm.at[b, pl.ds(pos0, PW)], idx_v.at[b], isem)
        for b in range(BATCH):
            pltpu.make_async_copy(
                x_hbm.at[b, pl.ds(pos0, PW)], idx_v.at[b], isem).wait()
        pltpu.async_copy(pe_hbm.at[pl.ds(pos0, CHUNK)], pe_v, psem)
        pltpu.async_copy(table_hbm.at[idx_v.at[0, pl.ds(0, CHUNK)]],
                         row_vv.at[0], gsem)

        def step(t, _):
            ci = t >> 2
            b = t & 3
            p = t & 1
            pos = pos0 + ci * CHUNK

            @pl.when(t >= 1)
            def _():
                pltpu.make_async_copy(
                    row_vv.at[1 - p],
                    out_hbm.at[(t - 1) & 3, pl.ds(pos0 + ((t - 1) >> 2) * CHUNK, CHUNK)],
                    ssem).wait()

            pltpu.make_async_copy(
                table_hbm.at[idx_v.at[b, pl.ds(ci * CHUNK, CHUNK)]],
                row_vv.at[p], gsem).wait()

            @pl.when(t < STEPS - 1)
            def _():
                nt = t + 1
                pltpu.async_copy(
                    table_hbm.at[idx_v.at[nt & 3, pl.ds((nt >> 2) * CHUNK, CHUNK)]],
                    row_vv.at[1 - p], gsem)

            @pl.when(b == 0)
            def _():
                pltpu.make_async_copy(
                    pe_hbm.at[pl.ds(pos, CHUNK)], pe_v, psem).wait()

            def row_body(r, _):
                for j in range(VECS_PER_ROW):
                    sl = pl.ds(j * LANES, LANES)
                    plsc.addupdate(row_vv.at[p, r, sl], pe_v[r, sl])
                return 0

            lax.fori_loop(0, CHUNK, row_body, 0)

            pltpu.async_copy(row_vv.at[p],
                             out_hbm.at[b, pl.ds(pos, CHUNK)], ssem)

            @pl.when((b == 3) & (t < STEPS - 1))
            def _():
                pltpu.async_copy(
                    pe_hbm.at[pl.ds(pos + CHUNK, CHUNK)], pe_v, psem)

            return 0

        lax.fori_loop(0, STEPS, step, 0)

        pltpu.make_async_copy(
            row_vv.at[(STEPS - 1) & 1],
            out_hbm.at[3, pl.ds(pos0 + PW - CHUNK, CHUNK)], ssem).wait()

    return body


_sc_kernel = _make_sc_kernel()


def kernel(x, table):
    pe = _sinusoidal_pe(SEQ_LEN, EMBED_DIM)
    return _sc_kernel(x.astype(jnp.int32), pe, table)


# speedup vs baseline: 1.6668x; 1.6668x over previous
import functools
import math

import jax
import jax.numpy as jnp
import numpy as np
from jax import lax
from jax.experimental import pallas as pl
from jax.experimental.pallas import tpu as pltpu
from jax.experimental.pallas import tpu_sc as plsc

VOCAB = 100000
EMBED_DIM = 2048
BATCH = 4
SEQ_LEN = 4096

NC, NS, LANES = 2, 16, 16
NW = NC * NS
PW = SEQ_LEN // NW
CHUNK = 16
STEPS = (PW // CHUNK) * BATCH
VECS_PER_ROW = EMBED_DIM // LANES


def _sinusoidal_pe(seq_len: int, d: int):
    pos = np.arange(seq_len, dtype=np.float32)[:, None]
    div = np.exp(np.arange(0, d, 2, dtype=np.float32) * (-math.log(10000.0) / d))
    pe = np.zeros((seq_len, d), dtype=np.float32)
    pe[:, 0::2] = np.sin(pos * div)
    pe[:, 1::2] = np.cos(pos * div)
    return pe


_PE = _sinusoidal_pe(SEQ_LEN, EMBED_DIM)


def _make_sc_kernel():
    mesh = plsc.VectorSubcoreMesh(
        core_axis_name="c", subcore_axis_name="s",
        num_cores=NC, num_subcores=NS,
    )

    @functools.partial(
        pl.kernel,
        out_type=jax.ShapeDtypeStruct((BATCH, SEQ_LEN, EMBED_DIM), jnp.float32),
        mesh=mesh,
        scratch_types=[
            pltpu.VMEM((BATCH, PW), jnp.int32),
            pltpu.VMEM((CHUNK, EMBED_DIM), jnp.float32),
            pltpu.VMEM((2, CHUNK, EMBED_DIM), jnp.float32),
            pltpu.SemaphoreType.DMA,
            pltpu.SemaphoreType.DMA,
            pltpu.SemaphoreType.DMA,
            pltpu.SemaphoreType.DMA,
        ],
    )
    def body(x_hbm, pe_hbm, table_hbm, out_hbm, idx_v, pe_v, row_vv,
             isem, gsem, ssem, psem):
        wid = lax.axis_index("s") * NC + lax.axis_index("c")
        pos0 = wid * PW

        for b in range(BATCH):
            pltpu.async_copy(x_hbm.at[b, pl.ds(pos0, PW)], idx_v.at[b], isem)
        for b in range(BATCH):
            pltpu.make_async_copy(
                x_hbm.at[b, pl.ds(pos0, PW)], idx_v.at[b], isem).wait()
        pltpu.async_copy(pe_hbm.at[pl.ds(pos0, CHUNK)], pe_v, psem)
        pltpu.async_copy(table_hbm.at[idx_v.at[0, pl.ds(0, CHUNK)]],
                         row_vv.at[0], gsem)

        def step(t, _):
            ci = t >> 2
            b = t & 3
            p = t & 1
            pos = pos0 + ci * CHUNK

            @pl.when(t >= 1)
            def _():
                pltpu.make_async_copy(
                    row_vv.at[1 - p],
                    out_hbm.at[(t - 1) & 3, pl.ds(pos0 + ((t - 1) >> 2) * CHUNK, CHUNK)],
                    ssem).wait()

            pltpu.make_async_copy(
                table_hbm.at[idx_v.at[b, pl.ds(ci * CHUNK, CHUNK)]],
                row_vv.at[p], gsem).wait()

            @pl.when(t < STEPS - 1)
            def _():
                nt = t + 1
                pltpu.async_copy(
                    table_hbm.at[idx_v.at[nt & 3, pl.ds((nt >> 2) * CHUNK, CHUNK)]],
                    row_vv.at[1 - p], gsem)

            @pl.when(b == 0)
            def _():
                pltpu.make_async_copy(
                    pe_hbm.at[pl.ds(pos, CHUNK)], pe_v, psem).wait()

            def row_body(r, _):
                for j in range(VECS_PER_ROW):
                    sl = pl.ds(j * LANES, LANES)
                    plsc.addupdate(row_vv.at[p, r, sl], pe_v[r, sl])
                return 0

            lax.fori_loop(0, CHUNK, row_body, 0)

            pltpu.async_copy(row_vv.at[p],
                             out_hbm.at[b, pl.ds(pos, CHUNK)], ssem)

            @pl.when((b == 3) & (t < STEPS - 1))
            def _():
                pltpu.async_copy(
                    pe_hbm.at[pl.ds(pos + CHUNK, CHUNK)], pe_v, psem)

            return 0

        lax.fori_loop(0, STEPS, step, 0)

        pltpu.make_async_copy(
            row_vv.at[(STEPS - 1) & 1],
            out_hbm.at[3, pl.ds(pos0 + PW - CHUNK, CHUNK)], ssem).wait()

    return body


_sc_kernel = _make_sc_kernel()


def kernel(x, table):
    return _sc_kernel(x.astype(jnp.int32), jnp.asarray(_PE), table)


# speedup vs baseline: 3.1092x; 1.8654x over previous
import functools
import math

import jax
import jax.numpy as jnp
import numpy as np
from jax import lax
from jax.experimental import pallas as pl
from jax.experimental.pallas import tpu as pltpu
from jax.experimental.pallas import tpu_sc as plsc

VOCAB = 100000
EMBED_DIM = 2048
BATCH = 4
SEQ_LEN = 4096

NC, NS, LANES = 2, 16, 16
NW = NC * NS
PW = SEQ_LEN // NW
CHUNK = 16
STEPS = (PW // CHUNK) * BATCH
VECS_PER_ROW = EMBED_DIM // LANES


def _sinusoidal_pe(seq_len: int, d: int):
    pos = np.arange(seq_len, dtype=np.float32)[:, None]
    div = np.exp(np.arange(0, d, 2, dtype=np.float32) * (-math.log(10000.0) / d))
    pe = np.zeros((seq_len, d), dtype=np.float32)
    pe[:, 0::2] = np.sin(pos * div)
    pe[:, 1::2] = np.cos(pos * div)
    return pe


_PE = _sinusoidal_pe(SEQ_LEN, EMBED_DIM)


def _make_sc_kernel():
    mesh = plsc.VectorSubcoreMesh(
        core_axis_name="c", subcore_axis_name="s",
        num_cores=NC, num_subcores=NS,
    )

    @functools.partial(
        pl.kernel,
        out_type=jax.ShapeDtypeStruct((BATCH, SEQ_LEN, EMBED_DIM), jnp.float32),
        mesh=mesh,
        scratch_types=[
            pltpu.VMEM((BATCH, PW), jnp.int32),
            pltpu.VMEM((CHUNK, EMBED_DIM), jnp.float32),
            pltpu.VMEM((2, CHUNK, EMBED_DIM), jnp.float32),
            pltpu.SemaphoreType.DMA,
            pltpu.SemaphoreType.DMA,
            pltpu.SemaphoreType.DMA,
            pltpu.SemaphoreType.DMA,
        ],
    )
    def body(x_hbm, pe_hbm, table_hbm, out_hbm, idx_v, pe_v, row_vv,
             isem, gsem, ssem, psem):
        wid = lax.axis_index("s") * NC + lax.axis_index("c")
        pos0 = wid * PW

        for b in range(BATCH):
            pltpu.async_copy(x_hbm.at[b, pl.ds(pos0, PW)], idx_v.at[b], isem)
        for b in range(BATCH):
            pltpu.make_async_copy(
                x_hbm.at[b, pl.ds(pos0, PW)], idx_v.at[b], isem).wait()
        pltpu.async_copy(pe_hbm.at[pl.ds(pos0, CHUNK)], pe_v, psem)
        pltpu.async_copy(table_hbm.at[idx_v.at[0, pl.ds(0, CHUNK)]],
                         row_vv.at[0], gsem)

        def step(t, _):
            ci = t >> 2
            b = t & 3
            p = t & 1
            pos = pos0 + ci * CHUNK

            @pl.when(t >= 1)
            def _():
                pltpu.make_async_copy(
                    row_vv.at[1 - p],
                    out_hbm.at[(t - 1) & 3, pl.ds(pos0 + ((t - 1) >> 2) * CHUNK, CHUNK)],
                    ssem).wait()

            pltpu.make_async_copy(
                table_hbm.at[idx_v.at[b, pl.ds(ci * CHUNK, CHUNK)]],
                row_vv.at[p], gsem).wait()

            @pl.when(t < STEPS - 1)
            def _():
                nt = t + 1
                pltpu.async_copy(
                    table_hbm.at[idx_v.at[nt & 3, pl.ds((nt >> 2) * CHUNK, CHUNK)]],
                    row_vv.at[1 - p], gsem)

            @pl.when(b == 0)
            def _():
                pltpu.make_async_copy(
                    pe_hbm.at[pl.ds(pos, CHUNK)], pe_v, psem).wait()

            @plsc.parallel_loop(0, CHUNK * VECS_PER_ROW, unroll=8)
            def _(i):
                r = i >> 7
                c = (i & (VECS_PER_ROW - 1)) * LANES
                sl = pl.ds(c, LANES)
                plsc.addupdate(row_vv.at[p, r, sl], pe_v[r, sl])

            pltpu.async_copy(row_vv.at[p],
                             out_hbm.at[b, pl.ds(pos, CHUNK)], ssem)

            @pl.when((b == 3) & (t < STEPS - 1))
            def _():
                pltpu.async_copy(
                    pe_hbm.at[pl.ds(pos + CHUNK, CHUNK)], pe_v, psem)

            return 0

        lax.fori_loop(0, STEPS, step, 0)

        pltpu.make_async_copy(
            row_vv.at[(STEPS - 1) & 1],
            out_hbm.at[3, pl.ds(pos0 + PW - CHUNK, CHUNK)], ssem).wait()

    return body


_sc_kernel = _make_sc_kernel()


def kernel(x, table):
    return _sc_kernel(x.astype(jnp.int32), jnp.asarray(_PE), table)


# speedup vs baseline: 3.1103x; 1.0004x over previous
import functools
import math

import jax
import jax.numpy as jnp
import numpy as np
from jax import lax
from jax.experimental import pallas as pl
from jax.experimental.pallas import tpu as pltpu
from jax.experimental.pallas import tpu_sc as plsc

VOCAB = 100000
EMBED_DIM = 2048
BATCH = 4
SEQ_LEN = 4096

NC, NS, LANES = 2, 16, 16
NW = NC * NS
PW = SEQ_LEN // NW
CHUNK = 16
STEPS = (PW // CHUNK) * BATCH
VECS_PER_ROW = EMBED_DIM // LANES


def _sinusoidal_pe(seq_len: int, d: int):
    pos = np.arange(seq_len, dtype=np.float32)[:, None]
    div = np.exp(np.arange(0, d, 2, dtype=np.float32) * (-math.log(10000.0) / d))
    pe = np.zeros((seq_len, d), dtype=np.float32)
    pe[:, 0::2] = np.sin(pos * div)
    pe[:, 1::2] = np.cos(pos * div)
    return pe


_PE = _sinusoidal_pe(SEQ_LEN, EMBED_DIM)


def _make_sc_kernel():
    mesh = plsc.VectorSubcoreMesh(
        core_axis_name="c", subcore_axis_name="s",
        num_cores=NC, num_subcores=NS,
    )

    @functools.partial(
        pl.kernel,
        out_type=jax.ShapeDtypeStruct((BATCH, SEQ_LEN, EMBED_DIM), jnp.float32),
        mesh=mesh,
        scratch_types=[
            pltpu.VMEM((BATCH, PW), jnp.int32),
            pltpu.VMEM((CHUNK * EMBED_DIM,), jnp.float32),
            pltpu.VMEM((2, CHUNK, EMBED_DIM), jnp.float32),
            pltpu.SemaphoreType.DMA,
            pltpu.SemaphoreType.DMA,
            pltpu.SemaphoreType.DMA,
            pltpu.SemaphoreType.DMA,
        ],
    )
    def body(x_hbm, pe_hbm, table_hbm, out_hbm, idx_v, pe_v, row_vv,
             isem, gsem, ssem, psem):
        wid = lax.axis_index("s") * NC + lax.axis_index("c")
        pos0 = wid * PW

        for b in range(BATCH):
            pltpu.async_copy(x_hbm.at[b, pl.ds(pos0, PW)], idx_v.at[b], isem)
        for b in range(BATCH):
            pltpu.make_async_copy(
                x_hbm.at[b, pl.ds(pos0, PW)], idx_v.at[b], isem).wait()
        pltpu.async_copy(
            pe_hbm.at[pl.ds(pos0 * EMBED_DIM, CHUNK * EMBED_DIM)], pe_v, psem)
        pltpu.async_copy(table_hbm.at[idx_v.at[0, pl.ds(0, CHUNK)]],
                         row_vv.at[0], gsem)

        def step(t, _):
            ci = t >> 2
            b = t & 3
            p = t & 1
            pos = pos0 + ci * CHUNK

            @pl.when(t >= 1)
            def _():
                pltpu.make_async_copy(
                    row_vv.at[1 - p],
                    out_hbm.at[(t - 1) & 3, pl.ds(pos0 + ((t - 1) >> 2) * CHUNK, CHUNK)],
                    ssem).wait()

            pltpu.make_async_copy(
                table_hbm.at[idx_v.at[b, pl.ds(ci * CHUNK, CHUNK)]],
                row_vv.at[p], gsem).wait()

            @pl.when(t < STEPS - 1)
            def _():
                nt = t + 1
                pltpu.async_copy(
                    table_hbm.at[idx_v.at[nt & 3, pl.ds((nt >> 2) * CHUNK, CHUNK)]],
                    row_vv.at[1 - p], gsem)

            @pl.when(b == 0)
            def _():
                pltpu.make_async_copy(
                    pe_hbm.at[pl.ds(pos * EMBED_DIM, CHUNK * EMBED_DIM)],
                    pe_v, psem).wait()

            @plsc.parallel_loop(0, CHUNK * VECS_PER_ROW, unroll=8)
            def _(i):
                r = i >> 7
                c = (i & (VECS_PER_ROW - 1)) * LANES
                plsc.addupdate(row_vv.at[p, r, pl.ds(c, LANES)],
                               pe_v[pl.ds(i * LANES, LANES)])

            pltpu.async_copy(row_vv.at[p],
                             out_hbm.at[b, pl.ds(pos, CHUNK)], ssem)

            @pl.when((b == 3) & (t < STEPS - 1))
            def _():
                pltpu.async_copy(
                    pe_hbm.at[pl.ds((pos + CHUNK) * EMBED_DIM,
                                    CHUNK * EMBED_DIM)], pe_v, psem)

            return 0

        lax.fori_loop(0, STEPS, step, 0)

        pltpu.make_async_copy(
            row_vv.at[(STEPS - 1) & 1],
            out_hbm.at[3, pl.ds(pos0 + PW - CHUNK, CHUNK)], ssem).wait()

    return body


_sc_kernel = _make_sc_kernel()


def kernel(x, table):
    return _sc_kernel(x.astype(jnp.int32), jnp.asarray(_PE.reshape(-1)), table)


# speedup vs baseline: 3.2237x; 1.0364x over previous
import functools
import math

import jax
import jax.numpy as jnp
import numpy as np
from jax import lax
from jax.experimental import pallas as pl
from jax.experimental.pallas import tpu as pltpu
from jax.experimental.pallas import tpu_sc as plsc

VOCAB = 100000
EMBED_DIM = 2048
BATCH = 4
SEQ_LEN = 4096

NC, NS, LANES = 2, 16, 16
NW = NC * NS
PW = SEQ_LEN // NW
CHUNK = 8
NB = 4
STEPS = (PW // CHUNK) * BATCH
VECS_PER_ROW = EMBED_DIM // LANES


def _sinusoidal_pe(seq_len: int, d: int):
    pos = np.arange(seq_len, dtype=np.float32)[:, None]
    div = np.exp(np.arange(0, d, 2, dtype=np.float32) * (-math.log(10000.0) / d))
    pe = np.zeros((seq_len, d), dtype=np.float32)
    pe[:, 0::2] = np.sin(pos * div)
    pe[:, 1::2] = np.cos(pos * div)
    return pe


_PE = _sinusoidal_pe(SEQ_LEN, EMBED_DIM)


def _make_sc_kernel():
    mesh = plsc.VectorSubcoreMesh(
        core_axis_name="c", subcore_axis_name="s",
        num_cores=NC, num_subcores=NS,
    )

    @functools.partial(
        pl.kernel,
        out_type=jax.ShapeDtypeStruct((BATCH, SEQ_LEN, EMBED_DIM), jnp.float32),
        mesh=mesh,
        scratch_types=[
            pltpu.VMEM((BATCH, PW), jnp.int32),
            pltpu.VMEM((CHUNK * EMBED_DIM,), jnp.float32),
            pltpu.VMEM((NB, CHUNK, EMBED_DIM), jnp.float32),
            pltpu.SemaphoreType.DMA,
            pltpu.SemaphoreType.DMA,
            pltpu.SemaphoreType.DMA((NB,)),
            pltpu.SemaphoreType.DMA,
        ],
    )
    def body(x_hbm, pe_hbm, table_hbm, out_hbm, idx_v, pe_v, row_vv,
             isem, gsem, ssems, psem):
        wid = lax.axis_index("s") * NC + lax.axis_index("c")
        pos0 = wid * PW

        for b in range(BATCH):
            pltpu.async_copy(x_hbm.at[b, pl.ds(pos0, PW)], idx_v.at[b], isem)
        for b in range(BATCH):
            pltpu.make_async_copy(
                x_hbm.at[b, pl.ds(pos0, PW)], idx_v.at[b], isem).wait()
        pltpu.async_copy(
            pe_hbm.at[pl.ds(pos0 * EMBED_DIM, CHUNK * EMBED_DIM)], pe_v, psem)
        pltpu.async_copy(table_hbm.at[idx_v.at[0, pl.ds(0, CHUNK)]],
                         row_vv.at[0], gsem)

        def step(t, _):
            ci = t >> 2
            b = t & 3
            q = t & (NB - 1)
            pos = pos0 + ci * CHUNK

            @pl.when(t >= NB - 1)
            def _():
                tp = t - (NB - 1)
                pltpu.make_async_copy(
                    row_vv.at[(tp & (NB - 1))],
                    out_hbm.at[tp & 3, pl.ds(pos0 + (tp >> 2) * CHUNK, CHUNK)],
                    ssems.at[tp & (NB - 1)]).wait()

            pltpu.make_async_copy(
                table_hbm.at[idx_v.at[b, pl.ds(ci * CHUNK, CHUNK)]],
                row_vv.at[q], gsem).wait()

            @pl.when(t < STEPS - 1)
            def _():
                nt = t + 1
                pltpu.async_copy(
                    table_hbm.at[idx_v.at[nt & 3, pl.ds((nt >> 2) * CHUNK, CHUNK)]],
                    row_vv.at[nt & (NB - 1)], gsem)

            @pl.when(b == 0)
            def _():
                pltpu.make_async_copy(
                    pe_hbm.at[pl.ds(pos * EMBED_DIM, CHUNK * EMBED_DIM)],
                    pe_v, psem).wait()

            @plsc.parallel_loop(0, CHUNK * VECS_PER_ROW, unroll=8)
            def _(i):
                r = i >> 7
                c = (i & (VECS_PER_ROW - 1)) * LANES
                plsc.addupdate(row_vv.at[q, r, pl.ds(c, LANES)],
                               pe_v[pl.ds(i * LANES, LANES)])

            pltpu.async_copy(row_vv.at[q],
                             out_hbm.at[b, pl.ds(pos, CHUNK)], ssems.at[q])

            @pl.when((b == 3) & (t < STEPS - 1))
            def _():
                pltpu.async_copy(
                    pe_hbm.at[pl.ds((pos + CHUNK) * EMBED_DIM,
                                    CHUNK * EMBED_DIM)], pe_v, psem)

            return 0

        lax.fori_loop(0, STEPS, step, 0)

        for k in range(NB - 1, 0, -1):
            tp = STEPS - k
            pltpu.make_async_copy(
                row_vv.at[tp & (NB - 1)],
                out_hbm.at[tp & 3,
                           pl.ds(pos0 + (tp >> 2) * CHUNK, CHUNK)],
                ssems.at[tp & (NB - 1)]).wait()

    return body


_sc_kernel = _make_sc_kernel()


def kernel(x, table):
    return _sc_kernel(x.astype(jnp.int32), jnp.asarray(_PE.reshape(-1)), table)


# speedup vs baseline: 3.2275x; 1.0012x over previous
import functools
import math

import jax
import jax.numpy as jnp
import numpy as np
from jax import lax
from jax.experimental import pallas as pl
from jax.experimental.pallas import tpu as pltpu
from jax.experimental.pallas import tpu_sc as plsc

VOCAB = 100000
EMBED_DIM = 2048
BATCH = 4
SEQ_LEN = 4096

NC, NS, LANES = 2, 16, 16
NW = NC * NS
PW = SEQ_LEN // NW
CHUNK = 8
NB = 4
STEPS = (PW // CHUNK) * BATCH
VECS_PER_ROW = EMBED_DIM // LANES


def _sinusoidal_pe(seq_len: int, d: int):
    pos = np.arange(seq_len, dtype=np.float32)[:, None]
    div = np.exp(np.arange(0, d, 2, dtype=np.float32) * (-math.log(10000.0) / d))
    pe = np.zeros((seq_len, d), dtype=np.float32)
    pe[:, 0::2] = np.sin(pos * div)
    pe[:, 1::2] = np.cos(pos * div)
    return pe


_PE = _sinusoidal_pe(SEQ_LEN, EMBED_DIM)


def _make_sc_kernel():
    mesh = plsc.VectorSubcoreMesh(
        core_axis_name="c", subcore_axis_name="s",
        num_cores=NC, num_subcores=NS,
    )

    @functools.partial(
        pl.kernel,
        out_type=jax.ShapeDtypeStruct((BATCH, SEQ_LEN, EMBED_DIM), jnp.float32),
        mesh=mesh,
        scratch_types=[
            pltpu.VMEM((BATCH, PW), jnp.int32),
            pltpu.VMEM((CHUNK * EMBED_DIM,), jnp.float32),
            pltpu.VMEM((NB, CHUNK, EMBED_DIM), jnp.float32),
            pltpu.SemaphoreType.DMA,
            pltpu.SemaphoreType.DMA,
            pltpu.SemaphoreType.DMA((NB,)),
            pltpu.SemaphoreType.DMA,
        ],
    )
    def body(x_hbm, pe_hbm, table_hbm, out_hbm, idx_v, pe_v, row_vv,
             isem, gsem, ssems, psem):
        wid = lax.axis_index("s") * NC + lax.axis_index("c")
        pos0 = wid * PW

        for b in range(BATCH):
            pltpu.async_copy(x_hbm.at[b, pl.ds(pos0, PW)], idx_v.at[b], isem)
        for b in range(BATCH):
            pltpu.make_async_copy(
                x_hbm.at[b, pl.ds(pos0, PW)], idx_v.at[b], isem).wait()
        pltpu.async_copy(
            pe_hbm.at[pl.ds(pos0 * EMBED_DIM, CHUNK * EMBED_DIM)], pe_v, psem)
        pltpu.async_copy(table_hbm.at[idx_v.at[0, pl.ds(0, CHUNK)]],
                         row_vv.at[0], gsem)

        def step(t, _):
            ci = t >> 2
            b = t & 3
            q = t & (NB - 1)
            pos = pos0 + ci * CHUNK

            @pl.when(t >= NB - 1)
            def _():
                tp = t - (NB - 1)
                pltpu.make_async_copy(
                    row_vv.at[(tp & (NB - 1))],
                    out_hbm.at[tp & 3, pl.ds(pos0 + (tp >> 2) * CHUNK, CHUNK)],
                    ssems.at[tp & (NB - 1)]).wait()

            pltpu.make_async_copy(
                table_hbm.at[idx_v.at[b, pl.ds(ci * CHUNK, CHUNK)]],
                row_vv.at[q], gsem).wait()

            @pl.when(t < STEPS - 1)
            def _():
                nt = t + 1
                pltpu.async_copy(
                    table_hbm.at[idx_v.at[nt & 3, pl.ds((nt >> 2) * CHUNK, CHUNK)]],
                    row_vv.at[nt & (NB - 1)], gsem)

            @pl.when(b == 0)
            def _():
                pltpu.make_async_copy(
                    pe_hbm.at[pl.ds(pos * EMBED_DIM, CHUNK * EMBED_DIM)],
                    pe_v, psem).wait()

            @plsc.parallel_loop(0, CHUNK * VECS_PER_ROW, unroll=16)
            def _(i):
                r = i >> 7
                c = (i & (VECS_PER_ROW - 1)) * LANES
                plsc.addupdate(row_vv.at[q, r, pl.ds(c, LANES)],
                               pe_v[pl.ds(i * LANES, LANES)])

            pltpu.async_copy(row_vv.at[q],
                             out_hbm.at[b, pl.ds(pos, CHUNK)], ssems.at[q])

            @pl.when((b == 3) & (t < STEPS - 1))
            def _():
                pltpu.async_copy(
                    pe_hbm.at[pl.ds((pos + CHUNK) * EMBED_DIM,
                                    CHUNK * EMBED_DIM)], pe_v, psem)

            return 0

        lax.fori_loop(0, STEPS, step, 0)

        for k in range(NB - 1, 0, -1):
            tp = STEPS - k
            pltpu.make_async_copy(
                row_vv.at[tp & (NB - 1)],
                out_hbm.at[tp & 3,
                           pl.ds(pos0 + (tp >> 2) * CHUNK, CHUNK)],
                ssems.at[tp & (NB - 1)]).wait()

    return body


_sc_kernel = _make_sc_kernel()


def kernel(x, table):
    return _sc_kernel(x.astype(jnp.int32), jnp.asarray(_PE.reshape(-1)), table)


# speedup vs baseline: 3.3346x; 1.0332x over previous
import functools
import math

import jax
import jax.numpy as jnp
import numpy as np
from jax import lax
from jax.experimental import pallas as pl
from jax.experimental.pallas import tpu as pltpu
from jax.experimental.pallas import tpu_sc as plsc

VOCAB = 100000
EMBED_DIM = 2048
BATCH = 4
SEQ_LEN = 4096

NC, NS, LANES = 2, 16, 16
NW = NC * NS
PW = SEQ_LEN // NW
CHUNK = 8
NB = 4
STEPS = (PW // CHUNK) * BATCH
VECS_PER_ROW = EMBED_DIM // LANES


def _sinusoidal_pe(seq_len: int, d: int):
    pos = np.arange(seq_len, dtype=np.float32)[:, None]
    div = np.exp(np.arange(0, d, 2, dtype=np.float32) * (-math.log(10000.0) / d))
    pe = np.zeros((seq_len, d), dtype=np.float32)
    pe[:, 0::2] = np.sin(pos * div)
    pe[:, 1::2] = np.cos(pos * div)
    return pe


_PE = _sinusoidal_pe(SEQ_LEN, EMBED_DIM)


def _make_sc_kernel():
    mesh = plsc.VectorSubcoreMesh(
        core_axis_name="c", subcore_axis_name="s",
        num_cores=NC, num_subcores=NS,
    )

    @functools.partial(
        pl.kernel,
        out_type=jax.ShapeDtypeStruct((BATCH, SEQ_LEN, EMBED_DIM), jnp.float32),
        mesh=mesh,
        scratch_types=[
            pltpu.VMEM((BATCH, PW), jnp.int32),
            pltpu.VMEM((CHUNK * EMBED_DIM,), jnp.float32),
            pltpu.VMEM((NB, CHUNK, EMBED_DIM), jnp.float32),
            pltpu.SemaphoreType.DMA,
            pltpu.SemaphoreType.DMA,
            pltpu.SemaphoreType.DMA((NB,)),
            pltpu.SemaphoreType.DMA,
        ],
    )
    def body(x_hbm, pe_hbm, table_hbm, out_hbm, idx_v, pe_v, row_vv,
             isem, gsem, ssems, psem):
        wid = lax.axis_index("s") * NC + lax.axis_index("c")
        pos0 = wid * PW

        for b in range(BATCH):
            pltpu.async_copy(x_hbm.at[b, pl.ds(pos0, PW)], idx_v.at[b], isem)
        for b in range(BATCH):
            pltpu.make_async_copy(
                x_hbm.at[b, pl.ds(pos0, PW)], idx_v.at[b], isem).wait()
        pltpu.async_copy(
            pe_hbm.at[pl.ds(pos0 * EMBED_DIM, CHUNK * EMBED_DIM)], pe_v, psem)
        pltpu.async_copy(table_hbm.at[idx_v.at[0, pl.ds(0, CHUNK)]],
                         row_vv.at[0], gsem)
        pltpu.async_copy(table_hbm.at[idx_v.at[1, pl.ds(0, CHUNK)]],
                         row_vv.at[1], gsem)

        def step(t, _):
            ci = t >> 2
            b = t & 3
            q = t & (NB - 1)
            pos = pos0 + ci * CHUNK

            @pl.when(t >= NB - 2)
            def _():
                tp = t - (NB - 2)
                pltpu.make_async_copy(
                    row_vv.at[(tp & (NB - 1))],
                    out_hbm.at[tp & 3, pl.ds(pos0 + (tp >> 2) * CHUNK, CHUNK)],
                    ssems.at[tp & (NB - 1)]).wait()

            pltpu.make_async_copy(
                table_hbm.at[idx_v.at[b, pl.ds(ci * CHUNK, CHUNK)]],
                row_vv.at[q], gsem).wait()

            @pl.when(t < STEPS - 2)
            def _():
                nt = t + 2
                pltpu.async_copy(
                    table_hbm.at[idx_v.at[nt & 3, pl.ds((nt >> 2) * CHUNK, CHUNK)]],
                    row_vv.at[nt & (NB - 1)], gsem)

            @pl.when(b == 0)
            def _():
                pltpu.make_async_copy(
                    pe_hbm.at[pl.ds(pos * EMBED_DIM, CHUNK * EMBED_DIM)],
                    pe_v, psem).wait()

            @plsc.parallel_loop(0, CHUNK * VECS_PER_ROW, unroll=16)
            def _(i):
                r = i >> 7
                c = (i & (VECS_PER_ROW - 1)) * LANES
                plsc.addupdate(row_vv.at[q, r, pl.ds(c, LANES)],
                               pe_v[pl.ds(i * LANES, LANES)])

            pltpu.async_copy(row_vv.at[q],
                             out_hbm.at[b, pl.ds(pos, CHUNK)], ssems.at[q])

            @pl.when((b == 3) & (t < STEPS - 1))
            def _():
                pltpu.async_copy(
                    pe_hbm.at[pl.ds((pos + CHUNK) * EMBED_DIM,
                                    CHUNK * EMBED_DIM)], pe_v, psem)

            return 0

        lax.fori_loop(0, STEPS, step, 0)

        for k in range(NB - 2, 0, -1):
            tp = STEPS - k
            pltpu.make_async_copy(
                row_vv.at[tp & (NB - 1)],
                out_hbm.at[tp & 3,
                           pl.ds(pos0 + (tp >> 2) * CHUNK, CHUNK)],
                ssems.at[tp & (NB - 1)]).wait()

    return body


_sc_kernel = _make_sc_kernel()


def kernel(x, table):
    return _sc_kernel(x.astype(jnp.int32), jnp.asarray(_PE.reshape(-1)), table)
